# baseline (speedup 1.0000x reference)
import math

import jax
import jax.numpy as jnp
from jax.experimental import pallas as pl

_B = 4
_E = 192
_R = 64
_N = _E + _R
_HSZ = 512
_RTOKS = 1000
_PROP = 2
_H = 4
_DH = _HSZ // _H

_FIELDS = ['Wq', 'Wk', 'Wv', 'Wo', 'W1', 'b1', 'W2', 'b2', 'a1',
           'ln1_g', 'ln1_b', 'ln2_g', 'ln2_b']


def _dot(a, b):
    return jax.lax.dot_general(
        a, b, (((1,), (0,)), ((), ())), preferred_element_type=jnp.float32
    )


def _bdot(a, b):
    return jax.lax.dot_general(
        a.astype(jnp.bfloat16), b.astype(jnp.bfloat16),
        (((1,), (0,)), ((), ())), preferred_element_type=jnp.float32
    )


def _layernorm(x, g, b, eps=1e-5):
    m = jnp.mean(x, axis=-1, keepdims=True)
    xc = x - m
    v = jnp.mean(xc * xc, axis=-1, keepdims=True)
    return xc * jax.lax.rsqrt(v + eps) * g + b


def _graph_kernel(rels_ref, vents_ref, adjs_ref, renc_ref, *refs):
    out_ref = refs[-1]
    prefs = refs[:-1]

    rels = rels_ref[0]
    ids = jnp.broadcast_to(rels.reshape(_R, 1), (_R, _RTOKS))
    iota = jax.lax.broadcasted_iota(jnp.int32, (_R, _RTOKS), 1)
    onehot = (ids == iota).astype(jnp.float32)
    vrel = _dot(onehot, renc_ref[...])

    vgraph = jnp.concatenate([vents_ref[0], vrel], axis=0)
    masked = adjs_ref[0] == 0.0
    scale = 1.0 / math.sqrt(_DH)

    nf = len(_FIELDS)
    for j in range(_PROP):
        p = dict(zip(_FIELDS, prefs[j * nf:(j + 1) * nf]))
        q = _bdot(vgraph, p['Wq'][...])
        k = _bdot(vgraph, p['Wk'][...])
        v = _bdot(vgraph, p['Wv'][...])
        outs = []
        for h in range(_H):
            sl = slice(h * _DH, (h + 1) * _DH)
            s = _bdot(q[:, sl], k[:, sl].T) * scale
            s = jnp.where(masked, -1e9, s)
            s = s - jnp.max(s, axis=-1, keepdims=True)
            e = jnp.exp(s)
            a = e / jnp.sum(e, axis=-1, keepdims=True)
            outs.append(_bdot(a, v[:, sl]))
        o = jnp.concatenate(outs, axis=-1)
        attn = _bdot(o, p['Wo'][...])
        t = _layernorm(attn, p['ln1_g'][0], p['ln1_b'][0])
        hdn = _bdot(t, p['W1'][...]) + p['b1'][0]
        hdn = jnp.where(hdn >= 0.0, hdn, p['a1'][0] * hdn)
        y = _bdot(hdn, p['W2'][...]) + p['b2'][0]
        vgraph = _layernorm(y + t, p['ln2_g'][0], p['ln2_b'][0])

    out_ref[0] = vgraph


@jax.jit
def _run(adjs, rels, vents, entlens, renc, params):
    rels3 = rels.astype(jnp.int32).reshape(_B, 1, _R)
    rep2 = lambda i: (0, 0)

    flat = []
    in_specs = [
        pl.BlockSpec((1, 1, _R), lambda i: (i, 0, 0)),
        pl.BlockSpec((1, _E, _HSZ), lambda i: (i, 0, 0)),
        pl.BlockSpec((1, _N, _N), lambda i: (i, 0, 0)),
        pl.BlockSpec((_RTOKS, _HSZ), rep2),
    ]
    for j in range(_PROP):
        for f in _FIELDS:
            w = params[j][f]
            if w.ndim == 1:
                w = w.reshape(1, -1)
            flat.append(w)
            in_specs.append(pl.BlockSpec(w.shape, rep2))

    gents = pl.pallas_call(
        _graph_kernel,
        grid=(_B,),
        in_specs=in_specs,
        out_specs=pl.BlockSpec((1, _N, _HSZ), lambda i: (i, 0, 0)),
        out_shape=jax.ShapeDtypeStruct((_B, _N, _HSZ), jnp.float32),
    )(rels3, vents, adjs, renc, *flat)

    globv = gents[:, _E, :]
    emask = jnp.arange(_N)[None, :] <= entlens[:, None]
    return (globv, gents, emask)


def kernel(adjs, rels, vents, entlens, renc, params):
    return _run(adjs, rels, vents, entlens, renc, params)


# speedup vs baseline: 1.0008x; 1.0008x over previous
import math

import jax
import jax.numpy as jnp
from jax.experimental import pallas as pl

_B = 4
_E = 192
_R = 64
_N = _E + _R
_HSZ = 512
_RTOKS = 1000
_PROP = 2
_H = 4
_DH = _HSZ // _H

_FIELDS = ['Wq', 'Wk', 'Wv', 'Wo', 'W1', 'b1', 'W2', 'b2', 'a1',
           'ln1_g', 'ln1_b', 'ln2_g', 'ln2_b']


def _dot(a, b):
    return jax.lax.dot_general(
        a, b, (((1,), (0,)), ((), ())), preferred_element_type=jnp.float32
    )


def _dot(a, b):
    return jax.lax.dot_general(
        a.astype(jnp.bfloat16), b.astype(jnp.bfloat16),
        (((1,), (0,)), ((), ())), preferred_element_type=jnp.float32
    )


def _layernorm(x, g, b, eps=1e-5):
    m = jnp.mean(x, axis=-1, keepdims=True)
    xc = x - m
    v = jnp.mean(xc * xc, axis=-1, keepdims=True)
    return xc * jax.lax.rsqrt(v + eps) * g + b


def _graph_kernel(rels_ref, vents_ref, adjs_ref, renc_ref, *refs):
    out_ref = refs[-1]
    prefs = refs[:-1]

    rels = rels_ref[0]
    ids = jnp.broadcast_to(rels.reshape(_R, 1), (_R, _RTOKS))
    iota = jax.lax.broadcasted_iota(jnp.int32, (_R, _RTOKS), 1)
    onehot = (ids == iota).astype(jnp.float32)
    vrel = _dot(onehot, renc_ref[...])

    vgraph = jnp.concatenate([vents_ref[0], vrel], axis=0)
    masked = adjs_ref[0] == 0.0
    scale = 1.0 / math.sqrt(_DH)

    nf = len(_FIELDS)
    for j in range(_PROP):
        p = dict(zip(_FIELDS, prefs[j * nf:(j + 1) * nf]))
        q = _dot(vgraph, p['Wq'][...])
        k = _dot(vgraph, p['Wk'][...])
        v = _dot(vgraph, p['Wv'][...])
        outs = []
        for h in range(_H):
            sl = slice(h * _DH, (h + 1) * _DH)
            s = _dot(q[:, sl], k[:, sl].T) * scale
            s = jnp.where(masked, -1e9, s)
            s = s - jnp.max(s, axis=-1, keepdims=True)
            e = jnp.exp(s)
            a = e / jnp.sum(e, axis=-1, keepdims=True)
            outs.append(_dot(a, v[:, sl]))
        o = jnp.concatenate(outs, axis=-1)
        attn = _dot(o, p['Wo'][...])
        t = _layernorm(attn, p['ln1_g'][0], p['ln1_b'][0])
        hdn = _dot(t, p['W1'][...]) + p['b1'][0]
        hdn = jnp.where(hdn >= 0.0, hdn, p['a1'][0] * hdn)
        y = _dot(hdn, p['W2'][...]) + p['b2'][0]
        vgraph = _layernorm(y + t, p['ln2_g'][0], p['ln2_b'][0])

    out_ref[0] = vgraph


@jax.jit
def _run(adjs, rels, vents, entlens, renc, params):
    rels3 = rels.astype(jnp.int32).reshape(_B, 1, _R)
    rep2 = lambda i: (0, 0)

    flat = []
    in_specs = [
        pl.BlockSpec((1, 1, _R), lambda i: (i, 0, 0)),
        pl.BlockSpec((1, _E, _HSZ), lambda i: (i, 0, 0)),
        pl.BlockSpec((1, _N, _N), lambda i: (i, 0, 0)),
        pl.BlockSpec((_RTOKS, _HSZ), rep2),
    ]
    for j in range(_PROP):
        for f in _FIELDS:
            w = params[j][f]
            if w.ndim == 1:
                w = w.reshape(1, -1)
            flat.append(w)
            in_specs.append(pl.BlockSpec(w.shape, rep2))

    gents = pl.pallas_call(
        _graph_kernel,
        grid=(_B,),
        in_specs=in_specs,
        out_specs=pl.BlockSpec((1, _N, _HSZ), lambda i: (i, 0, 0)),
        out_shape=jax.ShapeDtypeStruct((_B, _N, _HSZ), jnp.float32),
    )(rels3, vents, adjs, renc, *flat)

    globv = gents[:, _E, :]
    emask = jnp.arange(_N)[None, :] <= entlens[:, None]
    return (globv, gents, emask)


def kernel(adjs, rels, vents, entlens, renc, params):
    return _run(adjs, rels, vents, entlens, renc, params)


# speedup vs baseline: 1.2263x; 1.2253x over previous
import math

import jax
import jax.numpy as jnp
from jax.experimental import pallas as pl

_B = 4
_E = 192
_R = 64
_N = _E + _R
_HSZ = 512
_RTOKS = 1000
_PROP = 2
_H = 4
_DH = _HSZ // _H

_FIELDS = ['Wq', 'Wk', 'Wv', 'Wo', 'W1', 'b1', 'W2', 'b2', 'a1',
           'ln1_g', 'ln1_b', 'ln2_g', 'ln2_b']


def _dot(a, b):
    return jax.lax.dot_general(
        a, b, (((1,), (0,)), ((), ())), preferred_element_type=jnp.float32
    )


def _layernorm(x, g, b, eps=1e-5):
    m = jnp.mean(x, axis=-1, keepdims=True)
    xc = x - m
    v = jnp.mean(xc * xc, axis=-1, keepdims=True)
    return xc * jax.lax.rsqrt(v + eps) * g + b


def _graph_kernel(rels_ref, vents_ref, adjs_ref, renc_ref, *refs):
    out_ref = refs[-1]
    prefs = refs[:-1]

    rels = rels_ref[0]
    ids = jnp.broadcast_to(rels.reshape(_B * _R, 1), (_B * _R, _RTOKS))
    iota = jax.lax.broadcasted_iota(jnp.int32, (_B * _R, _RTOKS), 1)
    onehot = (ids == iota).astype(jnp.float32)
    vrel = _dot(onehot, renc_ref[...])

    pieces = []
    for i in range(_B):
        pieces.append(vents_ref[i])
        pieces.append(vrel[i * _R:(i + 1) * _R])
    vg = jnp.concatenate(pieces, axis=0)

    bias = jnp.where(adjs_ref[...] == 0.0, -1e9, 0.0)
    scale = 1.0 / math.sqrt(_DH)

    nf = len(_FIELDS)
    for j in range(_PROP):
        p = dict(zip(_FIELDS, prefs[j * nf:(j + 1) * nf]))
        q = _dot(vg, p['Wq'][...])
        k = _dot(vg, p['Wk'][...])
        v = _dot(vg, p['Wv'][...])
        outs = []
        for i in range(_B):
            rows = slice(i * _N, (i + 1) * _N)
            bi = bias[i]
            houts = []
            for h in range(_H):
                cols = slice(h * _DH, (h + 1) * _DH)
                s = _dot(q[rows, cols], k[rows, cols].T) * scale + bi
                s = s - jnp.max(s, axis=-1, keepdims=True)
                e = jnp.exp(s)
                r = 1.0 / jnp.sum(e, axis=-1, keepdims=True)
                houts.append(_dot(e, v[rows, cols]) * r)
            outs.append(jnp.concatenate(houts, axis=-1))
        o = jnp.concatenate(outs, axis=0)
        attn = _dot(o, p['Wo'][...])
        t = _layernorm(attn, p['ln1_g'][0], p['ln1_b'][0])
        hdn = _dot(t, p['W1'][...]) + p['b1'][0]
        hdn = jnp.where(hdn >= 0.0, hdn, p['a1'][0] * hdn)
        y = _dot(hdn, p['W2'][...]) + p['b2'][0]
        vg = _layernorm(y + t, p['ln2_g'][0], p['ln2_b'][0])

    out_ref[...] = vg.reshape(_B, _N, _HSZ)


@jax.jit
def _run(adjs, rels, vents, entlens, renc, params):
    rels2 = rels.astype(jnp.int32).reshape(1, _B * _R)
    rep2 = lambda: (0, 0)
    rep3 = lambda: (0, 0, 0)

    flat = []
    in_specs = [
        pl.BlockSpec((1, _B * _R), rep2),
        pl.BlockSpec((_B, _E, _HSZ), rep3),
        pl.BlockSpec((_B, _N, _N), rep3),
        pl.BlockSpec((_RTOKS, _HSZ), rep2),
    ]
    for j in range(_PROP):
        for f in _FIELDS:
            w = params[j][f]
            if w.ndim == 1:
                w = w.reshape(1, -1)
            flat.append(w)
            in_specs.append(pl.BlockSpec(w.shape, rep2))

    gents = pl.pallas_call(
        _graph_kernel,
        grid=(),
        in_specs=in_specs,
        out_specs=pl.BlockSpec((_B, _N, _HSZ), rep3),
        out_shape=jax.ShapeDtypeStruct((_B, _N, _HSZ), jnp.float32),
    )(rels2, vents, adjs, renc, *flat)

    globv = gents[:, _E, :]
    emask = jnp.arange(_N)[None, :] <= entlens[:, None]
    return (globv, gents, emask)


def kernel(adjs, rels, vents, entlens, renc, params):
    return _run(adjs, rels, vents, entlens, renc, params)


# speedup vs baseline: 1.3500x; 1.1009x over previous
import math

import jax
import jax.numpy as jnp
from jax.experimental import pallas as pl
from jax.experimental.pallas import tpu as pltpu

_B = 4
_E = 192
_R = 64
_N = _E + _R
_HSZ = 512
_RTOKS = 1000
_PROP = 2
_H = 4
_DH = _HSZ // _H

_BIG = ['Wq', 'Wk', 'Wv', 'Wo', 'W1', 'W2']
_SMALL = ['b1', 'b2', 'a1', 'ln1_g', 'ln1_b', 'ln2_g', 'ln2_b']
_NBIG = _PROP * len(_BIG)


def _dot(a, b):
    return jax.lax.dot_general(
        a, b, (((1,), (0,)), ((), ())), preferred_element_type=jnp.float32
    )


def _layernorm(x, g, b, eps=1e-5):
    m = jnp.mean(x, axis=-1, keepdims=True)
    xc = x - m
    v = jnp.mean(xc * xc, axis=-1, keepdims=True)
    return xc * jax.lax.rsqrt(v + eps) * g + b


def _graph_kernel(rels_ref, vents_ref, adjs_ref, renc_ref, *refs):
    hbm = refs[:_NBIG]
    small = refs[_NBIG:_NBIG + _PROP * len(_SMALL)]
    out_ref = refs[_NBIG + _PROP * len(_SMALL)]
    wbuf = refs[_NBIG + _PROP * len(_SMALL) + 1:-1]
    sems = refs[-1]

    for idx in range(_NBIG):
        pltpu.make_async_copy(hbm[idx], wbuf[idx], sems.at[idx]).start()

    def wg(j, name):
        idx = j * len(_BIG) + _BIG.index(name)
        pltpu.make_async_copy(hbm[idx], wbuf[idx], sems.at[idx]).wait()
        return wbuf[idx][...]

    def sm(j, name):
        return small[j * len(_SMALL) + _SMALL.index(name)][0]

    rels = rels_ref[0]
    ids = jnp.broadcast_to(rels.reshape(_B * _R, 1), (_B * _R, _RTOKS))
    iota = jax.lax.broadcasted_iota(jnp.int32, (_B * _R, _RTOKS), 1)
    onehot = (ids == iota).astype(jnp.float32)
    vrel = _dot(onehot, renc_ref[...])

    pieces = []
    for i in range(_B):
        pieces.append(vents_ref[i])
        pieces.append(vrel[i * _R:(i + 1) * _R])
    vg = jnp.concatenate(pieces, axis=0)

    bias = jnp.where(adjs_ref[...] == 0.0, -1e9, 0.0)
    scale = 1.0 / math.sqrt(_DH)

    for j in range(_PROP):
        q = _dot(vg, wg(j, 'Wq'))
        k = _dot(vg, wg(j, 'Wk'))
        v = _dot(vg, wg(j, 'Wv'))
        outs = []
        for i in range(_B):
            rows = slice(i * _N, (i + 1) * _N)
            bi = bias[i]
            houts = []
            for h in range(_H):
                cols = slice(h * _DH, (h + 1) * _DH)
                s = _dot(q[rows, cols], k[rows, cols].T) * scale + bi
                s = s - jnp.max(s, axis=-1, keepdims=True)
                e = jnp.exp(s)
                r = 1.0 / jnp.sum(e, axis=-1, keepdims=True)
                houts.append(_dot(e, v[rows, cols]) * r)
            outs.append(jnp.concatenate(houts, axis=-1))
        o = jnp.concatenate(outs, axis=0)
        attn = _dot(o, wg(j, 'Wo'))
        t = _layernorm(attn, sm(j, 'ln1_g'), sm(j, 'ln1_b'))
        hdn = _dot(t, wg(j, 'W1')) + sm(j, 'b1')
        hdn = jnp.where(hdn >= 0.0, hdn, sm(j, 'a1') * hdn)
        y = _dot(hdn, wg(j, 'W2')) + sm(j, 'b2')
        vg = _layernorm(y + t, sm(j, 'ln2_g'), sm(j, 'ln2_b'))

    out_ref[...] = vg.reshape(_B, _N, _HSZ)


@jax.jit
def _run(adjs, rels, vents, entlens, renc, params):
    rels2 = rels.astype(jnp.int32).reshape(1, _B * _R)
    rep2 = lambda: (0, 0)
    rep3 = lambda: (0, 0, 0)

    big = []
    big_specs = []
    scratch = [pltpu.SemaphoreType.DMA((_NBIG,))]
    wbufs = []
    for j in range(_PROP):
        for f in _BIG:
            w = params[j][f]
            big.append(w)
            big_specs.append(pl.BlockSpec(memory_space=pltpu.MemorySpace.HBM))
            wbufs.append(pltpu.VMEM(w.shape, w.dtype))

    smalls = []
    small_specs = []
    for j in range(_PROP):
        for f in _SMALL:
            w = params[j][f].reshape(1, -1)
            smalls.append(w)
            small_specs.append(pl.BlockSpec(w.shape, rep2))

    in_specs = [
        pl.BlockSpec((1, _B * _R), rep2),
        pl.BlockSpec((_B, _E, _HSZ), rep3),
        pl.BlockSpec((_B, _N, _N), rep3),
        pl.BlockSpec((_RTOKS, _HSZ), rep2),
    ] + big_specs + small_specs

    gents = pl.pallas_call(
        _graph_kernel,
        grid=(),
        in_specs=in_specs,
        out_specs=pl.BlockSpec((_B, _N, _HSZ), rep3),
        out_shape=jax.ShapeDtypeStruct((_B, _N, _HSZ), jnp.float32),
        scratch_shapes=wbufs + scratch,
    )(rels2, vents, adjs, renc, *big, *smalls)

    globv = gents[:, _E, :]
    emask = jnp.arange(_N)[None, :] <= entlens[:, None]
    return (globv, gents, emask)


def kernel(adjs, rels, vents, entlens, renc, params):
    return _run(adjs, rels, vents, entlens, renc, params)
